# baseline (speedup 1.0000x reference)
import functools

import jax
import jax.numpy as jnp
from jax import lax
from jax.experimental import pallas as pl
from jax.experimental.pallas import tpu as pltpu
from jax.experimental.pallas import tpu_sc as plsc

DIM = 64
BATCH = 16384
LANES = 16
NUM_CORES = 2
NUM_SUBCORES = 16
NUM_WORKERS = NUM_CORES * NUM_SUBCORES
B_PER_W = BATCH // NUM_WORKERS

_LOG1P_COEFS = (
    5.629329962175689e-07,
    0.9999574422836304,
    -0.49920639395713806,
    0.3269723653793335,
    -0.2228347212076187,
    0.13076335191726685,
    -0.05262395367026329,
    0.01011890172958374,
)


def _softplus16(x):
    u = jnp.exp(-jnp.abs(x))
    acc = jnp.full((LANES,), _LOG1P_COEFS[-1], dtype=jnp.float32)
    for c in _LOG1P_COEFS[-2::-1]:
        acc = acc * u + jnp.float32(c)
    return jnp.maximum(x, jnp.float32(0.0)) + acc


def _sc_body(idx_hbm, table_hbm, out_hbm, idx_v, rows_v, sem):
    wid = lax.axis_index("s") * NUM_CORES + lax.axis_index("c")
    base = wid * B_PER_W
    pltpu.sync_copy(idx_hbm.at[pl.ds(base, B_PER_W)], idx_v)
    pltpu.async_copy(table_hbm.at[idx_v], rows_v, sem).wait()

    def row_body(r, carry):
        for j in range(DIM // LANES):
            sl = pl.ds(j * LANES, LANES)
            rows_v[r, sl] = _softplus16(rows_v[r, sl])
        return carry

    lax.fori_loop(0, B_PER_W, row_body, 0)
    pltpu.sync_copy(rows_v, out_hbm.at[pl.ds(base, B_PER_W)])


def kernel(target_w_id, S):
    idx = target_w_id.astype(jnp.int32)
    mesh = plsc.VectorSubcoreMesh(core_axis_name="c", subcore_axis_name="s")
    run = pl.kernel(
        _sc_body,
        mesh=mesh,
        out_type=jax.ShapeDtypeStruct((BATCH, DIM), jnp.float32),
        scratch_types=[
            pltpu.VMEM((B_PER_W,), jnp.int32),
            pltpu.VMEM((B_PER_W, DIM), jnp.float32),
            pltpu.SemaphoreType.DMA,
        ],
        compiler_params=pltpu.CompilerParams(use_tc_tiling_on_sc=False),
    )
    return run(idx, S)
